# baseline (speedup 1.0000x reference)
import jax
import jax.numpy as jnp
from jax import lax
from jax.experimental import pallas as pl
from jax.experimental.pallas import tpu as pltpu
from jax.experimental.pallas import tpu_sc as plsc

N_NODES = 10000
N_PAD = 10240
D_IN = 128
D_OUT2 = 64
CHUNK = 64
SUP = 16
K_PT = 320
NC, NS = 2, 16
NW = NC * NS
E_PAD = NS * K_PT * CHUNK
ROWS_PT = N_PAD // NS
NBUF = 4
BM = 1000


def _sc_agg_kernel(d, with_deg):
  mesh = plsc.VectorSubcoreMesh(core_axis_name="c", subcore_axis_name="s")
  out_type = jax.ShapeDtypeStruct((NC, N_PAD, d), jnp.float32)
  scratch = [
      pltpu.VMEM((SUP, CHUNK), jnp.int32),
      pltpu.VMEM((SUP, CHUNK), jnp.int32),
      pltpu.VMEM((NBUF, CHUNK, d), jnp.float32),
      pltpu.VMEM_SHARED((N_PAD, d), jnp.float32),
      [pltpu.SemaphoreType.DMA] * NBUF,
      [pltpu.SemaphoreType.DMA] * NBUF,
  ]

  def body(src_hbm, dst_hbm, tbl_hbm, agg_hbm, src_v, dst_v, rows_v, agg_sh,
           gsems, ssems):
    cid = lax.axis_index("c")
    sid = lax.axis_index("s")

    zero16 = jnp.zeros((16,), jnp.float32)
    one16 = jnp.ones((16,), jnp.float32)
    core_active = with_deg or 0

    def zrow(i, _):
      for j in range(d // 16):
        rows_v[0, i, pl.ds(j * 16, 16)] = zero16
      return 0
    lax.fori_loop(0, CHUNK, zrow, 0)
    r0 = sid * ROWS_PT

    @pl.when(jnp.logical_or(cid == 0, core_active == 1))
    def _():
      for k in range(ROWS_PT // CHUNK):
        pltpu.sync_copy(rows_v.at[0], agg_sh.at[pl.ds(r0 + k * CHUNK, CHUNK)])

    plsc.subcore_barrier()

    LOOKAHEAD = NBUF - 1
    base = sid * K_PT
    n_sup = K_PT // SUP

    def load_idx(s, ref, hbm):
      pltpu.sync_copy(hbm.at[pl.ds(base + s * SUP, SUP)], ref)

    def start_gather(idx_row, buf):
      pltpu.async_copy(tbl_hbm.at[src_v.at[idx_row]], rows_v.at[buf],
                       gsems[buf])

    def wait_gather(buf):
      pltpu.make_async_copy(tbl_hbm.at[src_v.at[0]], rows_v.at[buf],
                            gsems[buf]).wait()

    def start_scatter(idx_row, buf):
      pltpu.async_copy(rows_v.at[buf], agg_sh.at[dst_v.at[idx_row]],
                       ssems[buf], add=True)

    def wait_scatter(buf):
      pltpu.make_async_copy(rows_v.at[buf], agg_sh.at[dst_v.at[0]],
                            ssems[buf]).wait()

    @pl.when(cid == 0)
    def _():
      load_idx(0, src_v, src_hbm)
      load_idx(0, dst_v, dst_hbm)
      for b in range(LOOKAHEAD):
        start_gather(b, b)

      def sup_body(s, _):
        for j in range(SUP):
          buf = j % NBUF
          wait_gather(buf)
          start_scatter(j, buf)
          if j < SUP - LOOKAHEAD:
            nbuf_ = (j + LOOKAHEAD) % NBUF
            if j == 0:
              @pl.when(s > 0)
              def _():
                wait_scatter(nbuf_)
            else:
              wait_scatter(nbuf_)
            start_gather(j + LOOKAHEAD, nbuf_)

        @pl.when(s < n_sup - 1)
        def _():
          load_idx(s + 1, src_v, src_hbm)
          load_idx(s + 1, dst_v, dst_hbm)
          for b in range(LOOKAHEAD):
            wait_scatter(b)
            start_gather(b, b)
        return 0

      lax.fori_loop(0, n_sup, sup_body, 0)
      for b in range(NBUF):
        wait_scatter(b)

    if with_deg:
      @pl.when(cid == 1)
      def _():
        def orow(i, _):
          for j in range(d // 16):
            rows_v[0, i, pl.ds(j * 16, 16)] = one16
          return 0
        lax.fori_loop(0, CHUNK, orow, 0)

        def deg_sup(s, _):
          load_idx(s, dst_v, dst_hbm)
          for j in range(SUP):
            pltpu.sync_copy(rows_v.at[0], agg_sh.at[dst_v.at[j]], add=True)
          return 0
        lax.fori_loop(0, n_sup, deg_sup, 0)

    plsc.subcore_barrier()

    @pl.when(jnp.logical_or(cid == 0, core_active == 1))
    def _():
      pltpu.sync_copy(agg_sh.at[pl.ds(r0, ROWS_PT)],
                      agg_hbm.at[cid, pl.ds(r0, ROWS_PT)])

  return pl.kernel(body, out_type=out_type, mesh=mesh, scratch_types=scratch)


def _tc1_body(x_ref, a_ref, dg_ref, ws1_ref, wn1_ref, b1_ref, h1_ref):
  inv = 1.0 / jnp.maximum(dg_ref[:, 0:1], 1.0)
  mean = a_ref[...] * inv
  h1 = x_ref[...] @ ws1_ref[...] + mean @ wn1_ref[...] + b1_ref[...]
  h1_ref[...] = jnp.maximum(h1, 0.0)


def _tc2_body(h1_ref, a_ref, dg_ref, ws2_ref, wn2_ref, b2_ref, out_ref):
  inv = 1.0 / jnp.maximum(dg_ref[:, 0:1], 1.0)
  mean = a_ref[...] * inv
  out_ref[...] = (h1_ref[...] @ ws2_ref[...] + mean @ wn2_ref[...]
                  + b2_ref[...])


def _row_spec(w):
  return pl.BlockSpec((BM, w), lambda i: (i, 0))


def _full_spec(h, w):
  return pl.BlockSpec((h, w), lambda i: (0, 0))


_tc1 = pl.pallas_call(
    _tc1_body,
    grid=(N_NODES // BM,),
    in_specs=[
        _row_spec(D_IN), _row_spec(D_IN), _row_spec(D_IN),
        _full_spec(D_IN, D_IN), _full_spec(D_IN, D_IN), _full_spec(1, D_IN),
    ],
    out_specs=_row_spec(D_IN),
    out_shape=jax.ShapeDtypeStruct((N_NODES, D_IN), jnp.float32),
)

_tc2 = pl.pallas_call(
    _tc2_body,
    grid=(N_NODES // BM,),
    in_specs=[
        _row_spec(D_IN), _row_spec(D_IN), _row_spec(D_IN),
        _full_spec(D_IN, D_OUT2), _full_spec(D_IN, D_OUT2),
        _full_spec(1, D_OUT2),
    ],
    out_specs=_row_spec(D_OUT2),
    out_shape=jax.ShapeDtypeStruct((N_NODES, D_OUT2), jnp.float32),
)

_agg_deg = _sc_agg_kernel(D_IN, True)
_agg_only = _sc_agg_kernel(D_IN, False)


def kernel(x, edge_index, W_self1, W_neigh1, b1, W_self2, W_neigh2, b2):
  e = edge_index.shape[1]
  pad = E_PAD - e
  src = jnp.concatenate(
      [edge_index[0], jnp.zeros((pad,), jnp.int32)]).reshape(-1, CHUNK)
  dst = jnp.concatenate(
      [edge_index[1], jnp.full((pad,), N_NODES, jnp.int32)]).reshape(-1, CHUNK)

  agg1 = _agg_deg(src, dst, x)
  a1, dg = agg1[0, :N_NODES], agg1[1, :N_NODES]

  h1 = _tc1(x, a1, dg, W_self1, W_neigh1, b1.reshape(1, -1))

  agg2 = _agg_only(src, dst, h1)
  out = _tc2(h1, agg2[0, :N_NODES], dg, W_self2, W_neigh2, b2.reshape(1, -1))
  return out


# speedup vs baseline: 1.0106x; 1.0106x over previous
import jax
import jax.numpy as jnp
from jax import lax
from jax.experimental import pallas as pl
from jax.experimental.pallas import tpu as pltpu
from jax.experimental.pallas import tpu_sc as plsc

N_NODES = 10000
N_PAD = 10240
D_IN = 128
D_OUT2 = 64
CHUNK = 64
SUP = 16
NC, NS = 2, 16
NW = NC * NS
K_AVG = 160
K0 = 208
K1 = 112
TOT_ROWS = NS * (K0 + K1)
E_PAD = TOT_ROWS * CHUNK
NCOPY = 4
ROWS_PT = N_PAD // NS
NBUF = 4
BM = 1000


def _sc_deg_kernel():
  mesh = plsc.VectorSubcoreMesh(core_axis_name="c", subcore_axis_name="s")
  out_type = jax.ShapeDtypeStruct((NC, N_PAD, D_IN), jnp.float32)
  scratch = [
      pltpu.VMEM((SUP, CHUNK), jnp.int32),
      pltpu.VMEM((CHUNK, D_IN), jnp.float32),
      pltpu.VMEM_SHARED((N_PAD, D_IN), jnp.float32),
  ]

  def body(dst_hbm, deg_hbm, dst_v, ones_v, deg_sh):
    cid = lax.axis_index("c")
    sid = lax.axis_index("s")
    wid = sid * NC + cid

    zero16 = jnp.zeros((16,), jnp.float32)
    one16 = jnp.ones((16,), jnp.float32)

    def fill(i, _):
      for j in range(D_IN // 16):
        ones_v[i, pl.ds(j * 16, 16)] = zero16
      return 0
    lax.fori_loop(0, CHUNK, fill, 0)
    r0 = sid * ROWS_PT
    for k in range(ROWS_PT // CHUNK):
      pltpu.sync_copy(ones_v, deg_sh.at[pl.ds(r0 + k * CHUNK, CHUNK)])
    def fill1(i, _):
      for j in range(D_IN // 16):
        ones_v[i, pl.ds(j * 16, 16)] = one16
      return 0
    lax.fori_loop(0, CHUNK, fill1, 0)

    plsc.subcore_barrier()

    base = wid * K_AVG
    def sup_body(s, _):
      pltpu.sync_copy(dst_hbm.at[pl.ds(base + s * SUP, SUP)], dst_v)
      for j in range(SUP):
        pltpu.sync_copy(ones_v, deg_sh.at[dst_v.at[j]], add=True)
      return 0
    lax.fori_loop(0, K_AVG // SUP, sup_body, 0)

    plsc.subcore_barrier()
    pltpu.sync_copy(deg_sh.at[pl.ds(r0, ROWS_PT)],
                    deg_hbm.at[cid, pl.ds(r0, ROWS_PT)])

  return pl.kernel(body, out_type=out_type, mesh=mesh, scratch_types=scratch)


def _sc_agg_kernel(d):
  mesh = plsc.VectorSubcoreMesh(core_axis_name="c", subcore_axis_name="s")
  out_type = jax.ShapeDtypeStruct((NC, N_PAD, d), jnp.float32)
  scratch = [
      pltpu.VMEM((SUP, CHUNK), jnp.int32),
      pltpu.VMEM((SUP, CHUNK), jnp.int32),
      pltpu.VMEM((NBUF, CHUNK, d), jnp.float32),
      pltpu.VMEM_SHARED((N_PAD, d), jnp.float32),
      [pltpu.SemaphoreType.DMA] * NBUF,
      [pltpu.SemaphoreType.DMA] * NBUF,
  ]

  def body(src_hbm, dst_hbm, tbl_hbm, agg_hbm, src_v, dst_v, rows_v, agg_sh,
           gsems, ssems):
    cid = lax.axis_index("c")
    sid = lax.axis_index("s")

    zero16 = jnp.zeros((16,), jnp.float32)

    def zrow(i, _):
      for j in range(d // 16):
        rows_v[0, i, pl.ds(j * 16, 16)] = zero16
      return 0
    lax.fori_loop(0, CHUNK, zrow, 0)
    r0 = sid * ROWS_PT
    for k in range(ROWS_PT // CHUNK):
      pltpu.sync_copy(rows_v.at[0], agg_sh.at[pl.ds(r0 + k * CHUNK, CHUNK)])

    plsc.subcore_barrier()

    LOOKAHEAD = NBUF - 1

    def load_sup(base, s):
      pltpu.sync_copy(src_hbm.at[pl.ds(base + s * SUP, SUP)], src_v)
      pltpu.sync_copy(dst_hbm.at[pl.ds(base + s * SUP, SUP)], dst_v)

    def start_gather(idx_row, buf):
      pltpu.async_copy(tbl_hbm.at[src_v.at[idx_row]], rows_v.at[buf],
                       gsems[buf])

    def wait_gather(buf):
      pltpu.make_async_copy(tbl_hbm.at[src_v.at[0]], rows_v.at[buf],
                            gsems[buf]).wait()

    def start_scatter(idx_row, buf):
      pltpu.async_copy(rows_v.at[buf], agg_sh.at[dst_v.at[idx_row]],
                       ssems[buf], add=True)

    def wait_scatter(buf):
      pltpu.make_async_copy(rows_v.at[buf], agg_sh.at[dst_v.at[0]],
                            ssems[buf]).wait()

    def run_pipeline(base, n_sup):
      load_sup(base, 0)
      for b in range(LOOKAHEAD):
        start_gather(b, b)

      def sup_body(s, _):
        for j in range(SUP):
          buf = j % NBUF
          wait_gather(buf)
          start_scatter(j, buf)
          if j < SUP - LOOKAHEAD:
            nbuf_ = (j + LOOKAHEAD) % NBUF
            if j == 0:
              @pl.when(s > 0)
              def _():
                wait_scatter(nbuf_)
            else:
              wait_scatter(nbuf_)
            start_gather(j + LOOKAHEAD, nbuf_)

        @pl.when(s < n_sup - 1)
        def _():
          load_sup(base, s + 1)
          for b in range(LOOKAHEAD):
            wait_scatter(b)
            start_gather(b, b)
        return 0

      lax.fori_loop(0, n_sup, sup_body, 0)
      for b in range(NBUF):
        wait_scatter(b)

    @pl.when(cid == 0)
    def _():
      run_pipeline(sid * K0, K0 // SUP)

    @pl.when(cid == 1)
    def _():
      run_pipeline(NS * K0 + sid * K1, K1 // SUP)

    plsc.subcore_barrier()
    pltpu.sync_copy(agg_sh.at[pl.ds(r0, ROWS_PT)],
                    agg_hbm.at[cid, pl.ds(r0, ROWS_PT)])

  return pl.kernel(body, out_type=out_type, mesh=mesh, scratch_types=scratch)


def _tc1_body(x_ref, a0_ref, a1_ref, d0_ref, d1_ref, ws1_ref, wn1_ref,
              b1_ref, h1_ref):
  deg = d0_ref[:, 0:1] + d1_ref[:, 0:1]
  inv = 1.0 / jnp.maximum(deg, 1.0)
  mean = (a0_ref[...] + a1_ref[...]) * inv
  h1 = x_ref[...] @ ws1_ref[...] + mean @ wn1_ref[...] + b1_ref[...]
  h1_ref[...] = jnp.maximum(h1, 0.0)


def _tc2_body(h1_ref, a0_ref, a1_ref, d0_ref, d1_ref, ws2_ref, wn2_ref,
              b2_ref, out_ref):
  deg = d0_ref[:, 0:1] + d1_ref[:, 0:1]
  inv = 1.0 / jnp.maximum(deg, 1.0)
  mean = (a0_ref[...] + a1_ref[...]) * inv
  out_ref[...] = (h1_ref[...] @ ws2_ref[...] + mean @ wn2_ref[...]
                  + b2_ref[...])


def _row_spec(w):
  return pl.BlockSpec((BM, w), lambda i: (i, 0))


def _full_spec(h, w):
  return pl.BlockSpec((h, w), lambda i: (0, 0))


_tc1 = pl.pallas_call(
    _tc1_body,
    grid=(N_NODES // BM,),
    in_specs=[
        _row_spec(D_IN), _row_spec(D_IN), _row_spec(D_IN),
        _row_spec(D_IN), _row_spec(D_IN),
        _full_spec(D_IN, D_IN), _full_spec(D_IN, D_IN), _full_spec(1, D_IN),
    ],
    out_specs=_row_spec(D_IN),
    out_shape=jax.ShapeDtypeStruct((N_NODES, D_IN), jnp.float32),
)

_tc2 = pl.pallas_call(
    _tc2_body,
    grid=(N_NODES // BM,),
    in_specs=[
        _row_spec(D_IN), _row_spec(D_IN), _row_spec(D_IN),
        _row_spec(D_IN), _row_spec(D_IN),
        _full_spec(D_IN, D_OUT2), _full_spec(D_IN, D_OUT2),
        _full_spec(1, D_OUT2),
    ],
    out_specs=_row_spec(D_OUT2),
    out_shape=jax.ShapeDtypeStruct((N_NODES, D_OUT2), jnp.float32),
)

_deg_k = _sc_deg_kernel()
_agg128 = _sc_agg_kernel(D_IN)


def _tile_of_row():
  r = jnp.arange(TOT_ROWS, dtype=jnp.int32)
  tile = jnp.where(r < NS * K0, r // K0, NS + (r - NS * K0) // K1)
  return (tile % NCOPY) * N_NODES


_COPY_OFF = _tile_of_row()


def kernel(x, edge_index, W_self1, W_neigh1, b1, W_self2, W_neigh2, b2):
  e = edge_index.shape[1]
  pad = E_PAD - e
  src = jnp.concatenate(
      [edge_index[0], jnp.zeros((pad,), jnp.int32)]).reshape(-1, CHUNK)
  src = src + _COPY_OFF[:, None]
  dst = jnp.concatenate(
      [edge_index[1], jnp.full((pad,), N_NODES, jnp.int32)]).reshape(-1, CHUNK)

  xc = jnp.concatenate([x] * NCOPY, axis=0)
  deg = _deg_k(dst)
  agg1 = _agg128(src, dst, xc)
  a10, a11 = agg1[0, :N_NODES], agg1[1, :N_NODES]
  d0, d1 = deg[0, :N_NODES], deg[1, :N_NODES]

  h1 = _tc1(x, a10, a11, d0, d1, W_self1, W_neigh1, b1.reshape(1, -1))

  h1c = jnp.concatenate([h1] * NCOPY, axis=0)
  agg2 = _agg128(src, dst, h1c)
  out = _tc2(h1, agg2[0, :N_NODES], agg2[1, :N_NODES], d0, d1,
             W_self2, W_neigh2, b2.reshape(1, -1))
  return out


# speedup vs baseline: 1.4828x; 1.4672x over previous
import jax
import jax.numpy as jnp
from jax import lax
from jax.experimental import pallas as pl
from jax.experimental.pallas import tpu as pltpu
from jax.experimental.pallas import tpu_sc as plsc

N_NODES = 10000
N_PAD = 10240
D_IN = 128
D_OUT2 = 64
CHUNK = 64
SUP = 16
NC, NS = 2, 16
NW = NC * NS
K_AVG = 160
K0 = 256
K1 = 64
TOT_ROWS = NS * (K0 + K1)
E_PAD = TOT_ROWS * CHUNK
NCOPY = 4
ROWS_PT = N_PAD // NS
NBUF = 4
BM = 1000


def _sc_deg_kernel():
  mesh = plsc.VectorSubcoreMesh(core_axis_name="c", subcore_axis_name="s")
  out_type = jax.ShapeDtypeStruct((NC, N_PAD, D_IN), jnp.float32)
  scratch = [
      pltpu.VMEM((SUP, CHUNK), jnp.int32),
      pltpu.VMEM((CHUNK, D_IN), jnp.float32),
      pltpu.VMEM_SHARED((N_PAD, D_IN), jnp.float32),
  ]

  def body(dst_hbm, deg_hbm, dst_v, ones_v, deg_sh):
    cid = lax.axis_index("c")
    sid = lax.axis_index("s")
    wid = sid * NC + cid

    zero16 = jnp.zeros((16,), jnp.float32)
    one16 = jnp.ones((16,), jnp.float32)

    def fill(i, _):
      for j in range(D_IN // 16):
        ones_v[i, pl.ds(j * 16, 16)] = zero16
      return 0
    lax.fori_loop(0, CHUNK, fill, 0)
    r0 = sid * ROWS_PT
    for k in range(ROWS_PT // CHUNK):
      pltpu.sync_copy(ones_v, deg_sh.at[pl.ds(r0 + k * CHUNK, CHUNK)])
    def fill1(i, _):
      for j in range(D_IN // 16):
        ones_v[i, pl.ds(j * 16, 16)] = one16
      return 0
    lax.fori_loop(0, CHUNK, fill1, 0)

    plsc.subcore_barrier()

    base = wid * K_AVG
    def sup_body(s, _):
      pltpu.sync_copy(dst_hbm.at[pl.ds(base + s * SUP, SUP)], dst_v)
      for j in range(SUP):
        pltpu.sync_copy(ones_v, deg_sh.at[dst_v.at[j]], add=True)
      return 0
    lax.fori_loop(0, K_AVG // SUP, sup_body, 0)

    plsc.subcore_barrier()
    pltpu.sync_copy(deg_sh.at[pl.ds(r0, ROWS_PT)],
                    deg_hbm.at[cid, pl.ds(r0, ROWS_PT)])

  return pl.kernel(body, out_type=out_type, mesh=mesh, scratch_types=scratch)


def _sc_agg_kernel(d):
  mesh = plsc.VectorSubcoreMesh(core_axis_name="c", subcore_axis_name="s")
  out_type = jax.ShapeDtypeStruct((NC, N_PAD, d), jnp.float32)
  scratch = [
      pltpu.VMEM((SUP, CHUNK), jnp.int32),
      pltpu.VMEM((SUP, CHUNK), jnp.int32),
      pltpu.VMEM((NBUF, CHUNK, d), jnp.float32),
      pltpu.VMEM_SHARED((N_PAD, d), jnp.float32),
      [pltpu.SemaphoreType.DMA] * NBUF,
      [pltpu.SemaphoreType.DMA] * NBUF,
  ]

  def body(src_hbm, dst_hbm, tbl_hbm, agg_hbm, src_v, dst_v, rows_v, agg_sh,
           gsems, ssems):
    cid = lax.axis_index("c")
    sid = lax.axis_index("s")

    zero16 = jnp.zeros((16,), jnp.float32)

    def zrow(i, _):
      for j in range(d // 16):
        rows_v[0, i, pl.ds(j * 16, 16)] = zero16
      return 0
    lax.fori_loop(0, CHUNK, zrow, 0)
    r0 = sid * ROWS_PT
    for k in range(ROWS_PT // CHUNK):
      pltpu.sync_copy(rows_v.at[0], agg_sh.at[pl.ds(r0 + k * CHUNK, CHUNK)])

    plsc.subcore_barrier()

    LOOKAHEAD = NBUF - 1

    def load_sup(base, s):
      pltpu.sync_copy(src_hbm.at[pl.ds(base + s * SUP, SUP)], src_v)
      pltpu.sync_copy(dst_hbm.at[pl.ds(base + s * SUP, SUP)], dst_v)

    def start_gather(idx_row, buf):
      pltpu.async_copy(tbl_hbm.at[src_v.at[idx_row]], rows_v.at[buf],
                       gsems[buf])

    def wait_gather(buf):
      pltpu.make_async_copy(tbl_hbm.at[src_v.at[0]], rows_v.at[buf],
                            gsems[buf]).wait()

    def start_scatter(idx_row, buf):
      pltpu.async_copy(rows_v.at[buf], agg_sh.at[dst_v.at[idx_row]],
                       ssems[buf], add=True)

    def wait_scatter(buf):
      pltpu.make_async_copy(rows_v.at[buf], agg_sh.at[dst_v.at[0]],
                            ssems[buf]).wait()

    def run_pipeline(base, n_sup):
      load_sup(base, 0)
      for b in range(LOOKAHEAD):
        start_gather(b, b)

      def sup_body(s, _):
        for j in range(SUP):
          buf = j % NBUF
          wait_gather(buf)
          start_scatter(j, buf)
          if j < SUP - LOOKAHEAD:
            nbuf_ = (j + LOOKAHEAD) % NBUF
            if j == 0:
              @pl.when(s > 0)
              def _():
                wait_scatter(nbuf_)
            else:
              wait_scatter(nbuf_)
            start_gather(j + LOOKAHEAD, nbuf_)

        @pl.when(s < n_sup - 1)
        def _():
          load_sup(base, s + 1)
          for b in range(LOOKAHEAD):
            wait_scatter(b)
            start_gather(b, b)
        return 0

      lax.fori_loop(0, n_sup, sup_body, 0)
      for b in range(NBUF):
        wait_scatter(b)

    @pl.when(cid == 0)
    def _():
      run_pipeline(sid * K0, K0 // SUP)

    @pl.when(cid == 1)
    def _():
      run_pipeline(NS * K0 + sid * K1, K1 // SUP)

    plsc.subcore_barrier()
    pltpu.sync_copy(agg_sh.at[pl.ds(r0, ROWS_PT)],
                    agg_hbm.at[cid, pl.ds(r0, ROWS_PT)])

  return pl.kernel(body, out_type=out_type, mesh=mesh, scratch_types=scratch)


def _tc1_body(x_ref, a0_ref, a1_ref, d0_ref, d1_ref, ws1_ref, wn1_ref,
              b1_ref, h1_ref):
  deg = d0_ref[:, 0:1] + d1_ref[:, 0:1]
  inv = 1.0 / jnp.maximum(deg, 1.0)
  mean = (a0_ref[...] + a1_ref[...]) * inv
  h1 = x_ref[...] @ ws1_ref[...] + mean @ wn1_ref[...] + b1_ref[...]
  h1_ref[...] = jnp.maximum(h1, 0.0)


def _tc2_body(h1_ref, a0_ref, a1_ref, d0_ref, d1_ref, ws2_ref, wn2_ref,
              b2_ref, out_ref):
  deg = d0_ref[:, 0:1] + d1_ref[:, 0:1]
  inv = 1.0 / jnp.maximum(deg, 1.0)
  mean = (a0_ref[...] + a1_ref[...]) * inv
  out_ref[...] = (h1_ref[...] @ ws2_ref[...] + mean @ wn2_ref[...]
                  + b2_ref[...])


def _row_spec(w):
  return pl.BlockSpec((BM, w), lambda i: (i, 0))


def _full_spec(h, w):
  return pl.BlockSpec((h, w), lambda i: (0, 0))


_tc1 = pl.pallas_call(
    _tc1_body,
    grid=(N_NODES // BM,),
    in_specs=[
        _row_spec(D_IN), _row_spec(D_IN), _row_spec(D_IN),
        _row_spec(D_IN), _row_spec(D_IN),
        _full_spec(D_IN, D_IN), _full_spec(D_IN, D_IN), _full_spec(1, D_IN),
    ],
    out_specs=_row_spec(D_IN),
    out_shape=jax.ShapeDtypeStruct((N_NODES, D_IN), jnp.float32),
)

_tc2 = pl.pallas_call(
    _tc2_body,
    grid=(N_NODES // BM,),
    in_specs=[
        _row_spec(D_IN), _row_spec(D_IN), _row_spec(D_IN),
        _row_spec(D_IN), _row_spec(D_IN),
        _full_spec(D_IN, D_OUT2), _full_spec(D_IN, D_OUT2),
        _full_spec(1, D_OUT2),
    ],
    out_specs=_row_spec(D_OUT2),
    out_shape=jax.ShapeDtypeStruct((N_NODES, D_OUT2), jnp.float32),
)

_deg_k = _sc_deg_kernel()
_agg128 = _sc_agg_kernel(D_IN)


def _tile_of_row():
  r = jnp.arange(TOT_ROWS, dtype=jnp.int32)
  tile = jnp.where(r < NS * K0, r // K0, NS + (r - NS * K0) // K1)
  return (tile % NCOPY) * N_NODES


_COPY_OFF = _tile_of_row()


def kernel(x, edge_index, W_self1, W_neigh1, b1, W_self2, W_neigh2, b2):
  e = edge_index.shape[1]
  pad = E_PAD - e
  src = jnp.concatenate(
      [edge_index[0], jnp.zeros((pad,), jnp.int32)]).reshape(-1, CHUNK)
  src = src + _COPY_OFF[:, None]
  dst = jnp.concatenate(
      [edge_index[1], jnp.full((pad,), N_NODES, jnp.int32)]).reshape(-1, CHUNK)

  xc = jnp.concatenate([x] * NCOPY, axis=0)
  deg = _deg_k(dst)
  agg1 = _agg128(src, dst, xc)
  a10, a11 = agg1[0, :N_NODES], agg1[1, :N_NODES]
  d0, d1 = deg[0, :N_NODES], deg[1, :N_NODES]

  h1 = _tc1(x, a10, a11, d0, d1, W_self1, W_neigh1, b1.reshape(1, -1))

  h1c = jnp.concatenate([h1] * NCOPY, axis=0)
  agg2 = _agg128(src, dst, h1c)
  out = _tc2(h1, agg2[0, :N_NODES], agg2[1, :N_NODES], d0, d1,
             W_self2, W_neigh2, b2.reshape(1, -1))
  return out


# speedup vs baseline: 1.7899x; 1.2071x over previous
import jax
import jax.numpy as jnp
from jax import lax
from jax.experimental import pallas as pl
from jax.experimental.pallas import tpu as pltpu
from jax.experimental.pallas import tpu_sc as plsc

N_NODES = 10000
N_PAD = 10240
D_IN = 128
D_OUT2 = 64
CHUNK = 64
SUP = 16
NC, NS = 2, 16
NW = NC * NS
K_AVG = 160
K0 = 272
K1 = 48
TOT_ROWS = NS * (K0 + K1)
E_PAD = TOT_ROWS * CHUNK
NCOPY = 4
ROWS_PT = N_PAD // NS
NBUF = 4
BM = 1000


def _sc_deg_kernel():
  mesh = plsc.VectorSubcoreMesh(core_axis_name="c", subcore_axis_name="s")
  out_type = jax.ShapeDtypeStruct((NC, N_PAD, D_IN), jnp.float32)
  scratch = [
      pltpu.VMEM((SUP, CHUNK), jnp.int32),
      pltpu.VMEM((CHUNK, D_IN), jnp.float32),
      pltpu.VMEM_SHARED((N_PAD, D_IN), jnp.float32),
  ]

  def body(dst_hbm, deg_hbm, dst_v, ones_v, deg_sh):
    cid = lax.axis_index("c")
    sid = lax.axis_index("s")
    wid = sid * NC + cid

    zero16 = jnp.zeros((16,), jnp.float32)
    one16 = jnp.ones((16,), jnp.float32)

    def fill(i, _):
      for j in range(D_IN // 16):
        ones_v[i, pl.ds(j * 16, 16)] = zero16
      return 0
    lax.fori_loop(0, CHUNK, fill, 0)
    r0 = sid * ROWS_PT
    for k in range(ROWS_PT // CHUNK):
      pltpu.sync_copy(ones_v, deg_sh.at[pl.ds(r0 + k * CHUNK, CHUNK)])
    def fill1(i, _):
      for j in range(D_IN // 16):
        ones_v[i, pl.ds(j * 16, 16)] = one16
      return 0
    lax.fori_loop(0, CHUNK, fill1, 0)

    plsc.subcore_barrier()

    base = wid * K_AVG
    def sup_body(s, _):
      pltpu.sync_copy(dst_hbm.at[pl.ds(base + s * SUP, SUP)], dst_v)
      for j in range(SUP):
        pltpu.sync_copy(ones_v, deg_sh.at[dst_v.at[j]], add=True)
      return 0
    lax.fori_loop(0, K_AVG // SUP, sup_body, 0)

    plsc.subcore_barrier()
    pltpu.sync_copy(deg_sh.at[pl.ds(r0, ROWS_PT)],
                    deg_hbm.at[cid, pl.ds(r0, ROWS_PT)])

  return pl.kernel(body, out_type=out_type, mesh=mesh, scratch_types=scratch)


def _sc_agg_kernel(d):
  mesh = plsc.VectorSubcoreMesh(core_axis_name="c", subcore_axis_name="s")
  out_type = jax.ShapeDtypeStruct((NC, N_PAD, d), jnp.float32)
  scratch = [
      pltpu.VMEM((SUP, CHUNK), jnp.int32),
      pltpu.VMEM((SUP, CHUNK), jnp.int32),
      pltpu.VMEM((NBUF, CHUNK, d), jnp.float32),
      pltpu.VMEM_SHARED((N_PAD, d), jnp.float32),
      [pltpu.SemaphoreType.DMA] * NBUF,
      [pltpu.SemaphoreType.DMA] * NBUF,
  ]

  def body(src_hbm, dst_hbm, tbl_hbm, agg_hbm, src_v, dst_v, rows_v, agg_sh,
           gsems, ssems):
    cid = lax.axis_index("c")
    sid = lax.axis_index("s")

    zero16 = jnp.zeros((16,), jnp.float32)

    def zrow(i, _):
      for j in range(d // 16):
        rows_v[0, i, pl.ds(j * 16, 16)] = zero16
      return 0
    lax.fori_loop(0, CHUNK, zrow, 0)
    r0 = sid * ROWS_PT
    for k in range(ROWS_PT // CHUNK):
      pltpu.sync_copy(rows_v.at[0], agg_sh.at[pl.ds(r0 + k * CHUNK, CHUNK)])

    plsc.subcore_barrier()

    LOOKAHEAD = NBUF - 1

    def load_sup(base, s):
      pltpu.sync_copy(src_hbm.at[pl.ds(base + s * SUP, SUP)], src_v)
      pltpu.sync_copy(dst_hbm.at[pl.ds(base + s * SUP, SUP)], dst_v)

    def start_gather(idx_row, buf):
      pltpu.async_copy(tbl_hbm.at[src_v.at[idx_row]], rows_v.at[buf],
                       gsems[buf])

    def wait_gather(buf):
      pltpu.make_async_copy(tbl_hbm.at[src_v.at[0]], rows_v.at[buf],
                            gsems[buf]).wait()

    def start_scatter(idx_row, buf):
      pltpu.async_copy(rows_v.at[buf], agg_sh.at[dst_v.at[idx_row]],
                       ssems[buf], add=True)

    def wait_scatter(buf):
      pltpu.make_async_copy(rows_v.at[buf], agg_sh.at[dst_v.at[0]],
                            ssems[buf]).wait()

    def run_pipeline(base, n_sup):
      load_sup(base, 0)
      for b in range(LOOKAHEAD):
        start_gather(b, b)

      def sup_body(s, _):
        for j in range(SUP):
          buf = j % NBUF
          wait_gather(buf)
          start_scatter(j, buf)
          if j < SUP - LOOKAHEAD:
            nbuf_ = (j + LOOKAHEAD) % NBUF
            if j == 0:
              @pl.when(s > 0)
              def _():
                wait_scatter(nbuf_)
            else:
              wait_scatter(nbuf_)
            start_gather(j + LOOKAHEAD, nbuf_)

        @pl.when(s < n_sup - 1)
        def _():
          load_sup(base, s + 1)
          for b in range(LOOKAHEAD):
            wait_scatter(b)
            start_gather(b, b)
        return 0

      lax.fori_loop(0, n_sup, sup_body, 0)
      for b in range(NBUF):
        wait_scatter(b)

    @pl.when(cid == 0)
    def _():
      run_pipeline(sid * K0, K0 // SUP)

    @pl.when(cid == 1)
    def _():
      run_pipeline(NS * K0 + sid * K1, K1 // SUP)

    plsc.subcore_barrier()
    pltpu.sync_copy(agg_sh.at[pl.ds(r0, ROWS_PT)],
                    agg_hbm.at[cid, pl.ds(r0, ROWS_PT)])

  return pl.kernel(body, out_type=out_type, mesh=mesh, scratch_types=scratch)


def _tc1_body(x_ref, a0_ref, a1_ref, d0_ref, d1_ref, ws1_ref, wn1_ref,
              b1_ref, h1_ref):
  deg = d0_ref[:, 0:1] + d1_ref[:, 0:1]
  inv = 1.0 / jnp.maximum(deg, 1.0)
  mean = (a0_ref[...] + a1_ref[...]) * inv
  h1 = x_ref[...] @ ws1_ref[...] + mean @ wn1_ref[...] + b1_ref[...]
  h1_ref[...] = jnp.maximum(h1, 0.0)


def _tc2_body(h1_ref, a0_ref, a1_ref, d0_ref, d1_ref, ws2_ref, wn2_ref,
              b2_ref, out_ref):
  deg = d0_ref[:, 0:1] + d1_ref[:, 0:1]
  inv = 1.0 / jnp.maximum(deg, 1.0)
  mean = (a0_ref[...] + a1_ref[...]) * inv
  out_ref[...] = (h1_ref[...] @ ws2_ref[...] + mean @ wn2_ref[...]
                  + b2_ref[...])


def _row_spec(w):
  return pl.BlockSpec((BM, w), lambda i: (i, 0))


def _full_spec(h, w):
  return pl.BlockSpec((h, w), lambda i: (0, 0))


_tc1 = pl.pallas_call(
    _tc1_body,
    grid=(N_NODES // BM,),
    in_specs=[
        _row_spec(D_IN), _row_spec(D_IN), _row_spec(D_IN),
        _row_spec(D_IN), _row_spec(D_IN),
        _full_spec(D_IN, D_IN), _full_spec(D_IN, D_IN), _full_spec(1, D_IN),
    ],
    out_specs=_row_spec(D_IN),
    out_shape=jax.ShapeDtypeStruct((N_NODES, D_IN), jnp.float32),
)

_tc2 = pl.pallas_call(
    _tc2_body,
    grid=(N_NODES // BM,),
    in_specs=[
        _row_spec(D_IN), _row_spec(D_IN), _row_spec(D_IN),
        _row_spec(D_IN), _row_spec(D_IN),
        _full_spec(D_IN, D_OUT2), _full_spec(D_IN, D_OUT2),
        _full_spec(1, D_OUT2),
    ],
    out_specs=_row_spec(D_OUT2),
    out_shape=jax.ShapeDtypeStruct((N_NODES, D_OUT2), jnp.float32),
)

_deg_k = _sc_deg_kernel()
_agg128 = _sc_agg_kernel(D_IN)


def _tile_of_row():
  r = jnp.arange(TOT_ROWS, dtype=jnp.int32)
  tile = jnp.where(r < NS * K0, r // K0, NS + (r - NS * K0) // K1)
  return (tile % NCOPY) * N_NODES


_COPY_OFF = _tile_of_row()


def kernel(x, edge_index, W_self1, W_neigh1, b1, W_self2, W_neigh2, b2):
  e = edge_index.shape[1]
  pad = E_PAD - e
  src = jnp.concatenate(
      [edge_index[0], jnp.zeros((pad,), jnp.int32)]).reshape(-1, CHUNK)
  src = src + _COPY_OFF[:, None]
  dst = jnp.concatenate(
      [edge_index[1], jnp.full((pad,), N_NODES, jnp.int32)]).reshape(-1, CHUNK)

  xc = jnp.concatenate([x] * NCOPY, axis=0)
  deg = _deg_k(dst)
  agg1 = _agg128(src, dst, xc)
  a10, a11 = agg1[0, :N_NODES], agg1[1, :N_NODES]
  d0, d1 = deg[0, :N_NODES], deg[1, :N_NODES]

  h1 = _tc1(x, a10, a11, d0, d1, W_self1, W_neigh1, b1.reshape(1, -1))

  h1c = jnp.concatenate([h1] * NCOPY, axis=0)
  agg2 = _agg128(src, dst, h1c)
  out = _tc2(h1, agg2[0, :N_NODES], agg2[1, :N_NODES], d0, d1,
             W_self2, W_neigh2, b2.reshape(1, -1))
  return out
